# Initial kernel scaffold; baseline (speedup 1.0000x reference)
import functools

import jax
import jax.numpy as jnp
from jax import lax
from jax.experimental import pallas as pl
from jax.experimental.pallas import tpu as pltpu
from jax.experimental.pallas import tpu_sc as plsc

N = 10000
E = 320000
NE = E + N
F = 128
FA = 144
NW = 32
R = 256
EPAD = 335872
GT = 2 * EPAD
CPW_G = GT // (NW * R)
CPW_S = EPAD // (NW * R)
C = 40

_mesh = plsc.VectorSubcoreMesh(core_axis_name="c", subcore_axis_name="s")


@functools.partial(
    pl.kernel,
    mesh=_mesh,
    out_type=jax.ShapeDtypeStruct((GT, FA), jnp.float32),
    scratch_types=[
        pltpu.VMEM((R,), jnp.int32),
        pltpu.VMEM((R, FA), jnp.float32),
        pltpu.SemaphoreType.DMA,
    ],
)
def _sc_gather(z_hbm, gidx_hbm, out_hbm, idx_v, rows_v, sem):
    wid = lax.axis_index("s") * 2 + lax.axis_index("c")
    base = wid * (CPW_G * R)

    @pl.loop(0, CPW_G)
    def _(i):
        b = pl.multiple_of(base + i * R, R)
        pltpu.sync_copy(gidx_hbm.at[pl.ds(b, R)], idx_v)
        pltpu.async_copy(z_hbm.at[idx_v], rows_v, sem).wait()
        pltpu.sync_copy(rows_v, out_hbm.at[pl.ds(b, R)])


@functools.partial(
    pl.kernel,
    mesh=_mesh,
    out_type=jax.ShapeDtypeStruct((2 * N, FA), jnp.float32),
    scratch_types=[
        pltpu.VMEM((R,), jnp.int32),
        pltpu.VMEM((R, FA), jnp.float32),
        pltpu.VMEM_SHARED((N, FA), jnp.float32),
    ],
)
def _sc_scatter(gidx_hbm, we_hbm, out_hbm, idx_v, we_v, acc_sh):
    c = lax.axis_index("c")
    s = lax.axis_index("s")

    @pl.loop(0, R)
    def _(i):
        @pl.loop(0, FA, step=16)
        def _(j):
            we_v[i, pl.ds(j, 16)] = jnp.zeros((16,), jnp.float32)

    @pl.loop(0, 5)
    def _(i):
        pltpu.sync_copy(
            we_v.at[pl.ds(0, 125)],
            acc_sh.at[pl.ds(s * 625 + i * 125, 125)],
        )

    plsc.subcore_barrier()

    base = c * (EPAD // 2) + s * (CPW_S * R) + EPAD

    @pl.loop(0, CPW_S)
    def _(i):
        b = pl.multiple_of(base + i * R, R)
        pltpu.sync_copy(gidx_hbm.at[pl.ds(b, R)], idx_v)
        pltpu.sync_copy(we_hbm.at[pl.ds(b - EPAD, R)], we_v)
        pltpu.sync_copy(we_v, acc_sh.at[idx_v], add=True)

    plsc.subcore_barrier()

    pltpu.sync_copy(
        acc_sh.at[pl.ds(s * 625, 625)],
        out_hbm.at[pl.ds(c * N + s * 625, 625)],
    )


def _tc_prep_body(x_ref, w1t_ref, b1_ref, z_ref):
    x = x_ref[...]
    h = lax.dot(x, w1t_ref[...], precision=lax.Precision.HIGHEST)
    h = jnp.maximum(h + b1_ref[...], 0.0)
    nrm = jnp.sqrt(jnp.sum(h * h, axis=1, keepdims=True))
    xn = h / jnp.maximum(nrm, 1e-12)
    z_ref[:, :F] = xn
    z_ref[:, F:FA] = jnp.broadcast_to(nrm, (x.shape[0], FA - F))


def _tc_edge_body(beta_ref, zs_ref, zd_ref, we_ref):
    beta = beta_ref[0, 0]
    zs = zs_ref[...]
    zd = zd_ref[...]
    rows = zs.shape[0]
    dot = jnp.sum(zs[:, :F] * zd[:, :F], axis=1, keepdims=True)
    rid = lax.broadcasted_iota(jnp.int32, (rows, 1), 0) + pl.program_id(0) * rows
    ex = jnp.where(rid < NE, jnp.exp(beta * dot - jnp.abs(beta)), 0.0)
    we_ref[:, :F] = zs[:, :F] * (ex * zs[:, F:F + 1])
    we_ref[:, F:FA] = jnp.broadcast_to(ex, (rows, FA - F))


def _combine(ua0, ua1):
    u = ua0 + ua1
    return u[:, :F] / u[:, F:F + 1]


def _tc_mid_body(ua0_ref, ua1_ref, z_ref):
    h = _combine(ua0_ref[...], ua1_ref[...])
    nrm = jnp.sqrt(jnp.sum(h * h, axis=1, keepdims=True))
    xn = h / jnp.maximum(nrm, 1e-12)
    z_ref[:, :F] = xn
    z_ref[:, F:FA] = jnp.broadcast_to(nrm, (h.shape[0], FA - F))


def _tc_out_body(ua0_ref, ua1_ref, wot_ref, bo_ref, o_ref):
    h = _combine(ua0_ref[...], ua1_ref[...])
    logits = lax.dot(h, wot_ref[...], precision=lax.Precision.HIGHEST)
    logits = logits + bo_ref[...]
    m = jnp.max(logits, axis=1, keepdims=True)
    lse = jnp.log(jnp.sum(jnp.exp(logits - m), axis=1, keepdims=True)) + m
    o_ref[...] = logits - lse


_BN = 1000
_BE = 1024


def _tc_prep(x, w1t, b1):
    return pl.pallas_call(
        _tc_prep_body,
        grid=(N // _BN,),
        in_specs=[
            pl.BlockSpec((_BN, F), lambda i: (i, 0)),
            pl.BlockSpec((F, F), lambda i: (0, 0)),
            pl.BlockSpec((1, F), lambda i: (0, 0)),
        ],
        out_specs=pl.BlockSpec((_BN, FA), lambda i: (i, 0)),
        out_shape=jax.ShapeDtypeStruct((N, FA), jnp.float32),
    )(x, w1t, b1)


def _tc_edge(beta_arr, g):
    nblk = EPAD // _BE
    return pl.pallas_call(
        _tc_edge_body,
        grid=(nblk,),
        in_specs=[
            pl.BlockSpec((1, F), lambda i: (0, 0)),
            pl.BlockSpec((_BE, FA), lambda i: (i, 0)),
            pl.BlockSpec((_BE, FA), lambda i, n=nblk: (i + n, 0)),
        ],
        out_specs=pl.BlockSpec((_BE, FA), lambda i: (i, 0)),
        out_shape=jax.ShapeDtypeStruct((EPAD, FA), jnp.float32),
    )(beta_arr, g)


def _tc_mid(ua):
    return pl.pallas_call(
        _tc_mid_body,
        grid=(N // _BN,),
        in_specs=[
            pl.BlockSpec((_BN, FA), lambda i: (i, 0)),
            pl.BlockSpec((_BN, FA), lambda i: (i + N // _BN, 0)),
        ],
        out_specs=pl.BlockSpec((_BN, FA), lambda i: (i, 0)),
        out_shape=jax.ShapeDtypeStruct((N, FA), jnp.float32),
    )(ua)


def _tc_out(ua, wot, bo):
    return pl.pallas_call(
        _tc_out_body,
        grid=(N // _BN,),
        in_specs=[
            pl.BlockSpec((_BN, FA), lambda i: (i, 0)),
            pl.BlockSpec((_BN, FA), lambda i: (i + N // _BN, 0)),
            pl.BlockSpec((F, C), lambda i: (0, 0)),
            pl.BlockSpec((1, C), lambda i: (0, 0)),
        ],
        out_specs=pl.BlockSpec((_BN, C), lambda i: (i, 0)),
        out_shape=jax.ShapeDtypeStruct((N, C), jnp.float32),
    )(ua, wot, bo)


def kernel(x, edge_index, edge_weight, W1, b1, Wout, bout, beta2):
    del edge_weight
    loop = jnp.arange(N, dtype=jnp.int32)
    pad = jnp.arange(EPAD - NE, dtype=jnp.int32) % N
    srcp = jnp.concatenate([edge_index[0], loop, pad])
    dstp = jnp.concatenate([edge_index[1], loop, pad])
    gidx = jnp.concatenate([srcp, dstp])

    w1t = W1.T
    b1r = b1.reshape(1, F)
    wot = Wout.T
    bor = bout.reshape(1, C)
    one = jnp.ones((1, F), jnp.float32)
    beta_arr = jnp.broadcast_to(beta2.astype(jnp.float32).reshape(1, 1), (1, F))

    z = _tc_prep(x, w1t, b1r)
    for beta in (one, beta_arr):
        g = _sc_gather(z, gidx)
        we = _tc_edge(beta, g)
        ua = _sc_scatter(gidx, we)
        if beta is one:
            z = _tc_mid(ua)
    return _tc_out(ua, wot, bor)


# speedup vs baseline: 8.6349x; 8.6349x over previous
import functools

import jax
import jax.numpy as jnp
from jax import lax
from jax.experimental import pallas as pl
from jax.experimental.pallas import tpu as pltpu
from jax.experimental.pallas import tpu_sc as plsc

N = 10000
NP = 10240
E = 320000
NE = E + N
F = 128
NW = 32
R = 256
CPW = 41
EPAD = NW * CPW * R
GT = 2 * EPAD
C = 40


@functools.cache
def _mesh():
    return plsc.VectorSubcoreMesh(core_axis_name="c", subcore_axis_name="s")


@functools.cache
def _make_sc_gather():
    @functools.partial(
        pl.kernel,
        mesh=_mesh(),
        out_type=jax.ShapeDtypeStruct((GT, F), jnp.float32),
        scratch_types=[
            pltpu.VMEM((R,), jnp.int32),
            pltpu.VMEM((R, F), jnp.float32),
            pltpu.SemaphoreType.DMA,
        ],
    )
    def body(y_hbm, z_hbm, gidx_hbm, out_hbm, idx_v, rows_v, sem):
        wid = lax.axis_index("s") * 2 + lax.axis_index("c")
        base = wid * (CPW * R)

        @pl.loop(0, CPW)
        def _(i):
            b = pl.multiple_of(base + i * R, R)
            pltpu.sync_copy(gidx_hbm.at[pl.ds(b, R)], idx_v)
            pltpu.async_copy(y_hbm.at[idx_v], rows_v, sem).wait()
            pltpu.sync_copy(rows_v, out_hbm.at[pl.ds(b, R)])

        @pl.loop(0, CPW)
        def _(i):
            b = pl.multiple_of(EPAD + base + i * R, R)
            pltpu.sync_copy(gidx_hbm.at[pl.ds(b, R)], idx_v)
            pltpu.async_copy(z_hbm.at[idx_v], rows_v, sem).wait()
            pltpu.sync_copy(rows_v, out_hbm.at[pl.ds(b, R)])

    return body


def _sc_gather(y, z, gidx):
    return _make_sc_gather()(y, z, gidx)


@functools.cache
def _make_sc_scatter():
    @functools.partial(
        pl.kernel,
        mesh=_mesh(),
        out_type=[
            jax.ShapeDtypeStruct((2 * NP, F), jnp.float32),
            jax.ShapeDtypeStruct((2 * NP,), jnp.float32),
        ],
        scratch_types=[
            pltpu.VMEM((R,), jnp.int32),
            pltpu.VMEM((R,), jnp.float32),
            pltpu.VMEM((R, F), jnp.float32),
            pltpu.VMEM_SHARED((NP, F), jnp.float32),
            pltpu.VMEM_SHARED((NP,), jnp.float32),
        ],
    )
    def body(gidx_hbm, we_hbm, ex_hbm, ua_hbm, dd_hbm,
             idx_v, ex_v, we_v, acc_sh, dacc_sh):
        c = lax.axis_index("c")
        s = lax.axis_index("s")

        @pl.loop(0, R)
        def _(i):
            @pl.loop(0, F, step=16)
            def _(j):
                we_v[i, pl.ds(j, 16)] = jnp.zeros((16,), jnp.float32)

        @pl.loop(0, R, step=16)
        def _(i):
            ex_v[pl.ds(i, 16)] = jnp.zeros((16,), jnp.float32)

        row0 = s * 640
        for off, sz in ((0, 256), (256, 256), (512, 128)):
            pltpu.sync_copy(we_v.at[pl.ds(0, sz)],
                            acc_sh.at[pl.ds(row0 + off, sz)])
            pltpu.sync_copy(ex_v.at[pl.ds(0, sz)],
                            dacc_sh.at[pl.ds(row0 + off, sz)])

        plsc.subcore_barrier()

        base = (c * 16 + s) * (CPW * R)

        @pl.loop(0, CPW)
        def _(i):
            b = pl.multiple_of(base + i * R, R)
            pltpu.sync_copy(gidx_hbm.at[pl.ds(EPAD + b, R)], idx_v)
            pltpu.sync_copy(we_hbm.at[pl.ds(b, R)], we_v)
            pltpu.sync_copy(ex_hbm.at[pl.ds(b, R)], ex_v)
            pltpu.sync_copy(we_v, acc_sh.at[idx_v], add=True)
            pltpu.sync_copy(ex_v, dacc_sh.at[idx_v], add=True)

        plsc.subcore_barrier()

        pltpu.sync_copy(acc_sh.at[pl.ds(row0, 640)],
                        ua_hbm.at[pl.ds(c * NP + row0, 640)])
        pltpu.sync_copy(dacc_sh.at[pl.ds(row0, 640)],
                        dd_hbm.at[pl.ds(c * NP + row0, 640)])

    return body


def _sc_scatter(gidx, we, ex):
    return _make_sc_scatter()(gidx, we, ex)


def _norm_split(h):
    nrm = jnp.sqrt(jnp.sum(h * h, axis=1, keepdims=True))
    return h / jnp.maximum(nrm, 1e-12)


def _tc_prep_body(x_ref, w1t_ref, b1_ref, h_ref, xn_ref):
    h = lax.dot(x_ref[...], w1t_ref[...], precision=lax.Precision.HIGHEST)
    h = jnp.maximum(h + b1_ref[...], 0.0)
    h_ref[...] = h
    xn_ref[...] = _norm_split(h)


def _tc_edge_body(beta_ref, hs_ref, zd_ref, we_ref, ex_ref):
    beta = beta_ref[0, 0]
    hs = hs_ref[...]
    zd = zd_ref[...]
    rows = hs.shape[0]
    ns = jnp.sqrt(jnp.sum(hs * hs, axis=1, keepdims=True))
    xns = hs / jnp.maximum(ns, 1e-12)
    prod = xns * zd
    dotc = jnp.sum(prod, axis=1, keepdims=True)
    rid = lax.broadcasted_iota(jnp.int32, (rows, 1), 0) + pl.program_id(0) * rows
    exc = jnp.where(rid < NE, jnp.exp(beta * dotc - jnp.abs(beta)), 0.0)
    we_ref[...] = hs * exc
    dot1 = jnp.sum(prod, axis=1)
    rid1 = lax.broadcasted_iota(jnp.int32, (rows,), 0) + pl.program_id(0) * rows
    ex_ref[...] = jnp.where(rid1 < NE, jnp.exp(beta * dot1 - jnp.abs(beta)), 0.0)


def _combine(ua0, ua1, dd0_ref, dd1_ref):
    i = pl.program_id(0)
    d = dd0_ref[pl.ds(i * _BN, _BN)] + dd1_ref[pl.ds(i * _BN, _BN)]
    return (ua0 + ua1) / d[:, None]


def _tc_mid_body(ua0_ref, ua1_ref, dd0_ref, dd1_ref, h_ref, xn_ref):
    h = _combine(ua0_ref[...], ua1_ref[...], dd0_ref, dd1_ref)
    h_ref[...] = h
    xn_ref[...] = _norm_split(h)


def _tc_out_body(ua0_ref, ua1_ref, dd0_ref, dd1_ref, wot_ref, bo_ref, o_ref):
    h = _combine(ua0_ref[...], ua1_ref[...], dd0_ref, dd1_ref)
    logits = lax.dot(h, wot_ref[...], precision=lax.Precision.HIGHEST)
    logits = logits + bo_ref[...]
    m = jnp.max(logits, axis=1, keepdims=True)
    lse = jnp.log(jnp.sum(jnp.exp(logits - m), axis=1, keepdims=True)) + m
    o_ref[...] = logits - lse


_BN = 1024
_BE = 1024


def _tc_prep(x, w1t, b1):
    return pl.pallas_call(
        _tc_prep_body,
        grid=(NP // _BN,),
        in_specs=[
            pl.BlockSpec((_BN, F), lambda i: (i, 0)),
            pl.BlockSpec((F, F), lambda i: (0, 0)),
            pl.BlockSpec((1, F), lambda i: (0, 0)),
        ],
        out_specs=[
            pl.BlockSpec((_BN, F), lambda i: (i, 0)),
            pl.BlockSpec((_BN, F), lambda i: (i, 0)),
        ],
        out_shape=[
            jax.ShapeDtypeStruct((NP, F), jnp.float32),
            jax.ShapeDtypeStruct((NP, F), jnp.float32),
        ],
    )(x, w1t, b1)


def _tc_edge(beta_arr, g):
    nblk = EPAD // _BE
    return pl.pallas_call(
        _tc_edge_body,
        grid=(nblk,),
        in_specs=[
            pl.BlockSpec((1, F), lambda i: (0, 0)),
            pl.BlockSpec((_BE, F), lambda i: (i, 0)),
            pl.BlockSpec((_BE, F), lambda i, n=nblk: (i + n, 0)),
        ],
        out_specs=[
            pl.BlockSpec((_BE, F), lambda i: (i, 0)),
            pl.BlockSpec((_BE,), lambda i: (i,)),
        ],
        out_shape=[
            jax.ShapeDtypeStruct((EPAD, F), jnp.float32),
            jax.ShapeDtypeStruct((EPAD,), jnp.float32),
        ],
    )(beta_arr, g, g)


_NODE_SPECS = [
    pl.BlockSpec((_BN, F), lambda i: (i, 0)),
    pl.BlockSpec((_BN, F), lambda i: (i, 0)),
    pl.BlockSpec((NP,), lambda i: (0,)),
    pl.BlockSpec((NP,), lambda i: (0,)),
]


def _tc_mid(ua0, ua1, dd0, dd1):
    return pl.pallas_call(
        _tc_mid_body,
        grid=(NP // _BN,),
        in_specs=list(_NODE_SPECS),
        out_specs=[
            pl.BlockSpec((_BN, F), lambda i: (i, 0)),
            pl.BlockSpec((_BN, F), lambda i: (i, 0)),
        ],
        out_shape=[
            jax.ShapeDtypeStruct((NP, F), jnp.float32),
            jax.ShapeDtypeStruct((NP, F), jnp.float32),
        ],
    )(ua0, ua1, dd0, dd1)


def _tc_out(ua0, ua1, dd0, dd1, wot, bo):
    return pl.pallas_call(
        _tc_out_body,
        grid=(NP // _BN,),
        in_specs=list(_NODE_SPECS) + [
            pl.BlockSpec((F, C), lambda i: (0, 0)),
            pl.BlockSpec((1, C), lambda i: (0, 0)),
        ],
        out_specs=pl.BlockSpec((_BN, C), lambda i: (i, 0)),
        out_shape=jax.ShapeDtypeStruct((NP, C), jnp.float32),
    )(ua0, ua1, dd0, dd1, wot, bo)


def kernel(x, edge_index, edge_weight, W1, b1, Wout, bout, beta2):
    del edge_weight
    loop = jnp.arange(N, dtype=jnp.int32)
    pad = jnp.arange(EPAD - NE, dtype=jnp.int32) % N
    srcp = jnp.concatenate([edge_index[0], loop, pad])
    dstp = jnp.concatenate([edge_index[1], loop, pad])
    gidx = jnp.concatenate([srcp, dstp])

    w1t = W1.T
    b1r = b1.reshape(1, F)
    wot = Wout.T
    bor = bout.reshape(1, C)
    one = jnp.ones((1, F), jnp.float32)
    beta_arr = jnp.broadcast_to(beta2.astype(jnp.float32).reshape(1, 1), (1, F))

    h, xn = _tc_prep(x, w1t, b1r)
    for beta in (one, beta_arr):
        g = _sc_gather(h, xn, gidx)
        we, ex = _tc_edge(beta, g)
        ua, dd = _sc_scatter(gidx, we, ex)
        ua0, ua1 = ua[:NP], ua[NP:]
        dd0, dd1 = dd[:NP], dd[NP:]
        if beta is one:
            h, xn = _tc_mid(ua0, ua1, dd0, dd1)
    return _tc_out(ua0, ua1, dd0, dd1, wot, bor)[:N]


# speedup vs baseline: 10.5628x; 1.2233x over previous
import functools

import jax
import jax.numpy as jnp
from jax import lax
from jax.experimental import pallas as pl
from jax.experimental.pallas import tpu as pltpu
from jax.experimental.pallas import tpu_sc as plsc

N = 10000
NP = 10240
E = 320000
NE = E + N
F = 128
NW = 32
R = 128
CPW = 82
EPAD = NW * CPW * R
GT = 2 * EPAD
C = 40


@functools.cache
def _mesh():
    return plsc.VectorSubcoreMesh(core_axis_name="c", subcore_axis_name="s")


CPWP = 88


@functools.cache
def _make_sc_gather():
    @functools.partial(
        pl.kernel,
        mesh=_mesh(),
        out_type=jax.ShapeDtypeStruct((GT, F), jnp.float32),
        scratch_types=[
            pltpu.VMEM((2 * CPWP, R), jnp.int32),
            pltpu.VMEM((R, F), jnp.float32),
            pltpu.VMEM((R, F), jnp.float32),
            pltpu.SemaphoreType.DMA,
            pltpu.SemaphoreType.DMA,
        ],
    )
    def body(y_hbm, z_hbm, sidx_hbm, didx_hbm, out_hbm,
             idx_v, buf_a, buf_b, sem_a, sem_b):
        wid = lax.axis_index("s") * 2 + lax.axis_index("c")
        base = wid * (CPW * R)

        pltpu.sync_copy(sidx_hbm.at[pl.ds(wid * CPWP, CPWP)],
                        idx_v.at[pl.ds(0, CPWP)])
        pltpu.sync_copy(didx_hbm.at[pl.ds(wid * CPWP, CPWP)],
                        idx_v.at[pl.ds(CPWP, CPWP)])

        bufs = (buf_a, buf_b)
        sems = (sem_a, sem_b)
        for li, table in ((0, y_hbm), (1, z_hbm)):
            out0 = li * EPAD + base

            def start(ci, p, table=table, li=li):
                return pltpu.async_copy(
                    table.at[idx_v.at[li * CPWP + ci]], bufs[p], sems[p])

            pend = [start(0, 0), start(1, 1)]
            for ci in range(CPW):
                p = ci & 1
                pend[p].wait()
                pltpu.sync_copy(bufs[p], out_hbm.at[pl.ds(out0 + ci * R, R)])
                if ci + 2 < CPW:
                    pend[p] = start(ci + 2, p)

    return body


def _sc_gather(y, z, sidx, didx):
    return _make_sc_gather()(y, z, sidx, didx)


@functools.cache
def _make_sc_scatter():
    @functools.partial(
        pl.kernel,
        mesh=_mesh(),
        out_type=[
            jax.ShapeDtypeStruct((2 * NP, F), jnp.float32),
            jax.ShapeDtypeStruct((2 * NP,), jnp.float32),
        ],
        scratch_types=[
            pltpu.VMEM((CPWP, R), jnp.int32),
            pltpu.VMEM((R,), jnp.float32),
            pltpu.VMEM((R,), jnp.float32),
            pltpu.VMEM((R, F), jnp.float32),
            pltpu.VMEM((R, F), jnp.float32),
            pltpu.VMEM_SHARED((NP, F), jnp.float32),
            pltpu.VMEM_SHARED((NP,), jnp.float32),
            pltpu.SemaphoreType.DMA,
            pltpu.SemaphoreType.DMA,
        ],
    )
    def body(didx_hbm, we_hbm, ex_hbm, ua_hbm, dd_hbm,
             idx_v, ex_a, ex_b, we_a, we_b, acc_sh, dacc_sh, sem_a, sem_b):
        c = lax.axis_index("c")
        s = lax.axis_index("s")
        w = c * 16 + s

        @pl.loop(0, R)
        def _(i):
            @pl.loop(0, F, step=16)
            def _(j):
                we_a[i, pl.ds(j, 16)] = jnp.zeros((16,), jnp.float32)

        @pl.loop(0, R, step=16)
        def _(i):
            ex_a[pl.ds(i, 16)] = jnp.zeros((16,), jnp.float32)

        row0 = s * 640
        for off in (0, 128, 256, 384, 512):
            pltpu.sync_copy(we_a, acc_sh.at[pl.ds(row0 + off, R)])
            pltpu.sync_copy(ex_a, dacc_sh.at[pl.ds(row0 + off, R)])

        pltpu.sync_copy(didx_hbm.at[pl.ds(w * CPWP, CPWP)], idx_v)

        plsc.subcore_barrier()

        webufs = (we_a, we_b)
        exbufs = (ex_a, ex_b)
        sems = (sem_a, sem_b)

        def start(ci, p):
            b = pl.multiple_of((w * CPW + ci) * R, R)
            hw = pltpu.async_copy(we_hbm.at[pl.ds(b, R)], webufs[p], sems[p])
            hx = pltpu.async_copy(ex_hbm.at[pl.ds(b, R)], exbufs[p], sems[p])
            return hw, hx

        pend = [start(0, 0), start(1, 1)]
        for ci in range(CPW):
            p = ci & 1
            hw, hx = pend[p]
            hw.wait()
            hx.wait()
            pltpu.sync_copy(webufs[p], acc_sh.at[idx_v.at[ci]], add=True)
            pltpu.sync_copy(exbufs[p], dacc_sh.at[idx_v.at[ci]], add=True)
            if ci + 2 < CPW:
                pend[p] = start(ci + 2, p)

        plsc.subcore_barrier()

        pltpu.sync_copy(acc_sh.at[pl.ds(row0, 640)],
                        ua_hbm.at[pl.ds(c * NP + row0, 640)])
        pltpu.sync_copy(dacc_sh.at[pl.ds(row0, 640)],
                        dd_hbm.at[pl.ds(c * NP + row0, 640)])

    return body


def _sc_scatter(didx, we, ex):
    return _make_sc_scatter()(didx, we, ex)


def _norm_split(h):
    nrm = jnp.sqrt(jnp.sum(h * h, axis=1, keepdims=True))
    return h / jnp.maximum(nrm, 1e-12)


def _tc_prep_body(x_ref, w1t_ref, b1_ref, h_ref, xn_ref):
    h = lax.dot(x_ref[...], w1t_ref[...], precision=lax.Precision.HIGHEST)
    h = jnp.maximum(h + b1_ref[...], 0.0)
    h_ref[...] = h
    xn_ref[...] = _norm_split(h)


def _tc_edge_body(beta_ref, hs_ref, zd_ref, we_ref, ex_ref):
    beta = beta_ref[0, 0]
    hs = hs_ref[...]
    zd = zd_ref[...]
    rows = hs.shape[0]
    ns = jnp.sqrt(jnp.sum(hs * hs, axis=1, keepdims=True))
    xns = hs / jnp.maximum(ns, 1e-12)
    prod = xns * zd
    dotc = jnp.sum(prod, axis=1, keepdims=True)
    rid = lax.broadcasted_iota(jnp.int32, (rows, 1), 0) + pl.program_id(0) * rows
    exc = jnp.where(rid < NE, jnp.exp(beta * dotc - jnp.abs(beta)), 0.0)
    we_ref[...] = hs * exc
    dot1 = jnp.sum(prod, axis=1)
    rid1 = lax.broadcasted_iota(jnp.int32, (rows,), 0) + pl.program_id(0) * rows
    ex_ref[...] = jnp.where(rid1 < NE, jnp.exp(beta * dot1 - jnp.abs(beta)), 0.0)


def _combine(ua0, ua1, dd0_ref, dd1_ref):
    i = pl.program_id(0)
    d = dd0_ref[pl.ds(i * _BN, _BN)] + dd1_ref[pl.ds(i * _BN, _BN)]
    return (ua0 + ua1) / d[:, None]


def _tc_mid_body(ua0_ref, ua1_ref, dd0_ref, dd1_ref, h_ref, xn_ref):
    h = _combine(ua0_ref[...], ua1_ref[...], dd0_ref, dd1_ref)
    h_ref[...] = h
    xn_ref[...] = _norm_split(h)


def _tc_out_body(ua0_ref, ua1_ref, dd0_ref, dd1_ref, wot_ref, bo_ref, o_ref):
    h = _combine(ua0_ref[...], ua1_ref[...], dd0_ref, dd1_ref)
    logits = lax.dot(h, wot_ref[...], precision=lax.Precision.HIGHEST)
    logits = logits + bo_ref[...]
    m = jnp.max(logits, axis=1, keepdims=True)
    lse = jnp.log(jnp.sum(jnp.exp(logits - m), axis=1, keepdims=True)) + m
    o_ref[...] = logits - lse


_BN = 1024
_BE = 1024


def _tc_prep(x, w1t, b1):
    return pl.pallas_call(
        _tc_prep_body,
        grid=(NP // _BN,),
        in_specs=[
            pl.BlockSpec((_BN, F), lambda i: (i, 0)),
            pl.BlockSpec((F, F), lambda i: (0, 0)),
            pl.BlockSpec((1, F), lambda i: (0, 0)),
        ],
        out_specs=[
            pl.BlockSpec((_BN, F), lambda i: (i, 0)),
            pl.BlockSpec((_BN, F), lambda i: (i, 0)),
        ],
        out_shape=[
            jax.ShapeDtypeStruct((NP, F), jnp.float32),
            jax.ShapeDtypeStruct((NP, F), jnp.float32),
        ],
    )(x, w1t, b1)


def _tc_edge(beta_arr, g):
    nblk = EPAD // _BE
    return pl.pallas_call(
        _tc_edge_body,
        grid=(nblk,),
        in_specs=[
            pl.BlockSpec((1, F), lambda i: (0, 0)),
            pl.BlockSpec((_BE, F), lambda i: (i, 0)),
            pl.BlockSpec((_BE, F), lambda i, n=nblk: (i + n, 0)),
        ],
        out_specs=[
            pl.BlockSpec((_BE, F), lambda i: (i, 0)),
            pl.BlockSpec((_BE,), lambda i: (i,)),
        ],
        out_shape=[
            jax.ShapeDtypeStruct((EPAD, F), jnp.float32),
            jax.ShapeDtypeStruct((EPAD,), jnp.float32),
        ],
    )(beta_arr, g, g)


_NODE_SPECS = [
    pl.BlockSpec((_BN, F), lambda i: (i, 0)),
    pl.BlockSpec((_BN, F), lambda i: (i, 0)),
    pl.BlockSpec((NP,), lambda i: (0,)),
    pl.BlockSpec((NP,), lambda i: (0,)),
]


def _tc_mid(ua0, ua1, dd0, dd1):
    return pl.pallas_call(
        _tc_mid_body,
        grid=(NP // _BN,),
        in_specs=list(_NODE_SPECS),
        out_specs=[
            pl.BlockSpec((_BN, F), lambda i: (i, 0)),
            pl.BlockSpec((_BN, F), lambda i: (i, 0)),
        ],
        out_shape=[
            jax.ShapeDtypeStruct((NP, F), jnp.float32),
            jax.ShapeDtypeStruct((NP, F), jnp.float32),
        ],
    )(ua0, ua1, dd0, dd1)


def _tc_out(ua0, ua1, dd0, dd1, wot, bo):
    return pl.pallas_call(
        _tc_out_body,
        grid=(NP // _BN,),
        in_specs=list(_NODE_SPECS) + [
            pl.BlockSpec((F, C), lambda i: (0, 0)),
            pl.BlockSpec((1, C), lambda i: (0, 0)),
        ],
        out_specs=pl.BlockSpec((_BN, C), lambda i: (i, 0)),
        out_shape=jax.ShapeDtypeStruct((NP, C), jnp.float32),
    )(ua0, ua1, dd0, dd1, wot, bo)


def kernel(x, edge_index, edge_weight, W1, b1, Wout, bout, beta2):
    del edge_weight
    loop = jnp.arange(N, dtype=jnp.int32)
    pad = jnp.arange(EPAD - NE, dtype=jnp.int32) % N
    def worker_rows(idx1d):
        slab = idx1d.reshape(NW, CPW, R)
        slab = jnp.pad(slab, ((0, 0), (0, CPWP - CPW), (0, 0)))
        return slab.reshape(NW * CPWP, R)

    sidx = worker_rows(jnp.concatenate([edge_index[0], loop, pad]))
    didx = worker_rows(jnp.concatenate([edge_index[1], loop, pad]))

    w1t = W1.T
    b1r = b1.reshape(1, F)
    wot = Wout.T
    bor = bout.reshape(1, C)
    one = jnp.ones((1, F), jnp.float32)
    beta_arr = jnp.broadcast_to(beta2.astype(jnp.float32).reshape(1, 1), (1, F))

    h, xn = _tc_prep(x, w1t, b1r)
    for beta in (one, beta_arr):
        g = _sc_gather(h, xn, sidx, didx)
        we, ex = _tc_edge(beta, g)
        ua, dd = _sc_scatter(didx, we, ex)
        ua0, ua1 = ua[:NP], ua[NP:]
        dd0, dd1 = dd[:NP], dd[NP:]
        if beta is one:
            h, xn = _tc_mid(ua0, ua1, dd0, dd1)
    return _tc_out(ua0, ua1, dd0, dd1, wot, bor)[:N]


# speedup vs baseline: 11.8437x; 1.1213x over previous
import functools

import jax
import jax.numpy as jnp
from jax import lax
from jax.experimental import pallas as pl
from jax.experimental.pallas import tpu as pltpu
from jax.experimental.pallas import tpu_sc as plsc

N = 10000
NP = 10240
E = 320000
NE = E + N
F = 128
NW = 32
R = 128
CPW = 41
EPAD_H = NW * CPW * R
EPAD = 2 * EPAD_H
GT = 2 * EPAD_H
C = 40


@functools.cache
def _mesh():
    return plsc.VectorSubcoreMesh(core_axis_name="c", subcore_axis_name="s")


CPWP = 48


@functools.cache
def _make_sc_gather():
    @functools.partial(
        pl.kernel,
        mesh=_mesh(),
        out_type=jax.ShapeDtypeStruct((GT, F), jnp.float32),
        scratch_types=[
            pltpu.VMEM((2 * CPWP, R), jnp.int32),
            pltpu.VMEM((R, F), jnp.float32),
            pltpu.VMEM((R, F), jnp.float32),
            pltpu.SemaphoreType.DMA,
            pltpu.SemaphoreType.DMA,
        ],
    )
    def body(y_hbm, z_hbm, sidx_hbm, didx_hbm, out_hbm,
             idx_v, buf_a, buf_b, sem_a, sem_b):
        wid = lax.axis_index("s") * 2 + lax.axis_index("c")
        base = wid * (CPW * R)

        pltpu.sync_copy(sidx_hbm.at[pl.ds(wid * CPWP, CPWP)],
                        idx_v.at[pl.ds(0, CPWP)])
        pltpu.sync_copy(didx_hbm.at[pl.ds(wid * CPWP, CPWP)],
                        idx_v.at[pl.ds(CPWP, CPWP)])

        bufs = (buf_a, buf_b)
        sems = (sem_a, sem_b)
        for li, table in ((0, y_hbm), (1, z_hbm)):
            out0 = li * EPAD_H + base

            def start(ci, p, table=table, li=li):
                return pltpu.async_copy(
                    table.at[idx_v.at[li * CPWP + ci]], bufs[p], sems[p])

            pend = [start(0, 0), start(1, 1)]
            for ci in range(CPW):
                p = ci & 1
                pend[p].wait()
                pltpu.sync_copy(bufs[p], out_hbm.at[pl.ds(out0 + ci * R, R)])
                if ci + 2 < CPW:
                    pend[p] = start(ci + 2, p)

    return body


def _sc_gather(y, z, sidx, didx):
    return _make_sc_gather()(y, z, sidx, didx)


@functools.cache
def _make_sc_scatter():
    @functools.partial(
        pl.kernel,
        mesh=_mesh(),
        out_type=[
            jax.ShapeDtypeStruct((2 * NP, F), jnp.float32),
            jax.ShapeDtypeStruct((2 * NP,), jnp.float32),
        ],
        scratch_types=[
            pltpu.VMEM((CPWP, R), jnp.int32),
            pltpu.VMEM((R,), jnp.float32),
            pltpu.VMEM((R,), jnp.float32),
            pltpu.VMEM((R, F), jnp.float32),
            pltpu.VMEM((R, F), jnp.float32),
            pltpu.VMEM_SHARED((NP, F), jnp.float32),
            pltpu.VMEM_SHARED((NP,), jnp.float32),
            pltpu.SemaphoreType.DMA,
            pltpu.SemaphoreType.DMA,
        ],
    )
    def body(didx_hbm, we_hbm, ex_hbm, ua_hbm, dd_hbm,
             idx_v, ex_a, ex_b, we_a, we_b, acc_sh, dacc_sh, sem_a, sem_b):
        c = lax.axis_index("c")
        s = lax.axis_index("s")
        w = c * 16 + s

        @pl.loop(0, R)
        def _(i):
            @pl.loop(0, F, step=16)
            def _(j):
                we_a[i, pl.ds(j, 16)] = jnp.zeros((16,), jnp.float32)

        @pl.loop(0, R, step=16)
        def _(i):
            ex_a[pl.ds(i, 16)] = jnp.zeros((16,), jnp.float32)

        row0 = s * 640
        for off in (0, 128, 256, 384, 512):
            pltpu.sync_copy(we_a, acc_sh.at[pl.ds(row0 + off, R)])
            pltpu.sync_copy(ex_a, dacc_sh.at[pl.ds(row0 + off, R)])

        pltpu.sync_copy(didx_hbm.at[pl.ds(w * CPWP, CPWP)], idx_v)

        plsc.subcore_barrier()

        webufs = (we_a, we_b)
        exbufs = (ex_a, ex_b)
        sems = (sem_a, sem_b)

        def start(ci, p):
            b = pl.multiple_of((w * CPW + ci) * R, R)
            hw = pltpu.async_copy(we_hbm.at[pl.ds(b, R)], webufs[p], sems[p])
            hx = pltpu.async_copy(ex_hbm.at[pl.ds(b, R)], exbufs[p], sems[p])
            return hw, hx

        pend = [start(0, 0), start(1, 1)]
        for ci in range(CPW):
            p = ci & 1
            hw, hx = pend[p]
            hw.wait()
            hx.wait()
            pltpu.sync_copy(webufs[p], acc_sh.at[idx_v.at[ci]], add=True)
            pltpu.sync_copy(exbufs[p], dacc_sh.at[idx_v.at[ci]], add=True)
            if ci + 2 < CPW:
                pend[p] = start(ci + 2, p)

        plsc.subcore_barrier()

        pltpu.sync_copy(acc_sh.at[pl.ds(row0, 640)],
                        ua_hbm.at[pl.ds(c * NP + row0, 640)])
        pltpu.sync_copy(dacc_sh.at[pl.ds(row0, 640)],
                        dd_hbm.at[pl.ds(c * NP + row0, 640)])

    return body


def _sc_scatter(didx, we, ex):
    return _make_sc_scatter()(didx, we, ex)


def _norm_split(h):
    nrm = jnp.sqrt(jnp.sum(h * h, axis=1, keepdims=True))
    return h / jnp.maximum(nrm, 1e-12)


def _tc_prep_body(x_ref, w1t_ref, b1_ref, h_ref, xn_ref):
    h = lax.dot(x_ref[...], w1t_ref[...], precision=lax.Precision.HIGHEST)
    h = jnp.maximum(h + b1_ref[...], 0.0)
    h_ref[...] = h
    xn_ref[...] = _norm_split(h)


def _tc_edge_body(goff, beta_ref, hs_ref, zd_ref, we_ref, ex_ref):
    beta = beta_ref[0, 0]
    hs = hs_ref[...]
    zd = zd_ref[...]
    rows = hs.shape[0]
    ns = jnp.sqrt(jnp.sum(hs * hs, axis=1, keepdims=True))
    xns = hs / jnp.maximum(ns, 1e-12)
    prod = xns * zd
    dotc = jnp.sum(prod, axis=1, keepdims=True)
    rid = (lax.broadcasted_iota(jnp.int32, (rows, 1), 0)
           + pl.program_id(0) * rows + goff)
    exc = jnp.where(rid < NE, jnp.exp(beta * dotc - jnp.abs(beta)), 0.0)
    we_ref[...] = hs * exc
    dot1 = jnp.sum(prod, axis=1)
    rid1 = (lax.broadcasted_iota(jnp.int32, (rows,), 0)
            + pl.program_id(0) * rows + goff)
    ex_ref[...] = jnp.where(rid1 < NE, jnp.exp(beta * dot1 - jnp.abs(beta)), 0.0)


def _combine(uas, dds, i):
    d = sum(r[pl.ds(i * _BN, _BN)] for r in dds)
    return sum(u[...] for u in uas) / d[:, None]


def _tc_mid_body(ua0, ua1, ua2, ua3, dd0, dd1, dd2, dd3, h_ref, xn_ref):
    h = _combine((ua0, ua1, ua2, ua3), (dd0, dd1, dd2, dd3), pl.program_id(0))
    h_ref[...] = h
    xn_ref[...] = _norm_split(h)


def _tc_out_body(ua0, ua1, ua2, ua3, dd0, dd1, dd2, dd3,
                 wot_ref, bo_ref, o_ref):
    h = _combine((ua0, ua1, ua2, ua3), (dd0, dd1, dd2, dd3), pl.program_id(0))
    logits = lax.dot(h, wot_ref[...], precision=lax.Precision.HIGHEST)
    logits = logits + bo_ref[...]
    m = jnp.max(logits, axis=1, keepdims=True)
    lse = jnp.log(jnp.sum(jnp.exp(logits - m), axis=1, keepdims=True)) + m
    o_ref[...] = logits - lse


_BN = 1024
_BE = 1024


def _tc_prep(x, w1t, b1):
    return pl.pallas_call(
        _tc_prep_body,
        grid=(NP // _BN,),
        in_specs=[
            pl.BlockSpec((_BN, F), lambda i: (i, 0)),
            pl.BlockSpec((F, F), lambda i: (0, 0)),
            pl.BlockSpec((1, F), lambda i: (0, 0)),
        ],
        out_specs=[
            pl.BlockSpec((_BN, F), lambda i: (i, 0)),
            pl.BlockSpec((_BN, F), lambda i: (i, 0)),
        ],
        out_shape=[
            jax.ShapeDtypeStruct((NP, F), jnp.float32),
            jax.ShapeDtypeStruct((NP, F), jnp.float32),
        ],
    )(x, w1t, b1)


def _tc_edge(beta_arr, g, goff):
    nblk = EPAD_H // _BE
    return pl.pallas_call(
        functools.partial(_tc_edge_body, goff),
        grid=(nblk,),
        in_specs=[
            pl.BlockSpec((1, F), lambda i: (0, 0)),
            pl.BlockSpec((_BE, F), lambda i: (i, 0)),
            pl.BlockSpec((_BE, F), lambda i, n=nblk: (i + n, 0)),
        ],
        out_specs=[
            pl.BlockSpec((_BE, F), lambda i: (i, 0)),
            pl.BlockSpec((_BE,), lambda i: (i,)),
        ],
        out_shape=[
            jax.ShapeDtypeStruct((EPAD_H, F), jnp.float32),
            jax.ShapeDtypeStruct((EPAD_H,), jnp.float32),
        ],
    )(beta_arr, g, g)


_NODE_SPECS = (
    [pl.BlockSpec((_BN, F), lambda i: (i, 0))] * 4
    + [pl.BlockSpec((NP,), lambda i: (0,))] * 4
)


def _tc_mid(uas, dds):
    return pl.pallas_call(
        _tc_mid_body,
        grid=(NP // _BN,),
        in_specs=list(_NODE_SPECS),
        out_specs=[
            pl.BlockSpec((_BN, F), lambda i: (i, 0)),
            pl.BlockSpec((_BN, F), lambda i: (i, 0)),
        ],
        out_shape=[
            jax.ShapeDtypeStruct((NP, F), jnp.float32),
            jax.ShapeDtypeStruct((NP, F), jnp.float32),
        ],
    )(*uas, *dds)


def _tc_out(uas, dds, wot, bo):
    return pl.pallas_call(
        _tc_out_body,
        grid=(NP // _BN,),
        in_specs=list(_NODE_SPECS) + [
            pl.BlockSpec((F, C), lambda i: (0, 0)),
            pl.BlockSpec((1, C), lambda i: (0, 0)),
        ],
        out_specs=pl.BlockSpec((_BN, C), lambda i: (i, 0)),
        out_shape=jax.ShapeDtypeStruct((NP, C), jnp.float32),
    )(*uas, *dds, wot, bo)


def kernel(x, edge_index, edge_weight, W1, b1, Wout, bout, beta2):
    del edge_weight
    loop = jnp.arange(N, dtype=jnp.int32)
    pad = jnp.arange(EPAD - NE, dtype=jnp.int32) % N

    def worker_rows(idx1d):
        slab = idx1d.reshape(NW, CPW, R)
        slab = jnp.pad(slab, ((0, 0), (0, CPWP - CPW), (0, 0)))
        return slab.reshape(NW * CPWP, R)

    sfull = jnp.concatenate([edge_index[0], loop, pad])
    dfull = jnp.concatenate([edge_index[1], loop, pad])
    sidx = [worker_rows(sfull[:EPAD_H]), worker_rows(sfull[EPAD_H:])]
    didx = [worker_rows(dfull[:EPAD_H]), worker_rows(dfull[EPAD_H:])]

    w1t = W1.T
    b1r = b1.reshape(1, F)
    wot = Wout.T
    bor = bout.reshape(1, C)
    one = jnp.ones((1, F), jnp.float32)
    beta_arr = jnp.broadcast_to(beta2.astype(jnp.float32).reshape(1, 1), (1, F))

    h, xn = _tc_prep(x, w1t, b1r)
    for beta in (one, beta_arr):
        g0 = _sc_gather(h, xn, sidx[0], didx[0])
        g1 = _sc_gather(h, xn, sidx[1], didx[1])
        we0, ex0 = _tc_edge(beta, g0, 0)
        we1, ex1 = _tc_edge(beta, g1, EPAD_H)
        uaA, ddA = _sc_scatter(didx[0], we0, ex0)
        uaB, ddB = _sc_scatter(didx[1], we1, ex1)
        uas = (uaA[:NP], uaA[NP:], uaB[:NP], uaB[NP:])
        dds = (ddA[:NP], ddA[NP:], ddB[:NP], ddB[NP:])
        if beta is one:
            h, xn = _tc_mid(uas, dds)
    return _tc_out(uas, dds, wot, bor)[:N]
